# Initial kernel scaffold; baseline (speedup 1.0000x reference)
import jax
import jax.numpy as jnp
from jax.experimental import pallas as pl


def kernel(positions, node_features, edge_index, batch, ne_W1, ne_b1, ne_W2, ne_b2, ee_W1, ee_b1, ee_W2, ee_b2, mp0_msg_W1, mp0_msg_b1, mp0_msg_W2, mp0_msg_b2, mp0_upd_W1, mp0_upd_b1, mp0_upd_W2, mp0_upd_b2, mp1_msg_W1, mp1_msg_b1, mp1_msg_W2, mp1_msg_b2, mp1_upd_W1, mp1_upd_b1, mp1_upd_W2, mp1_upd_b2, mp2_msg_W1, mp2_msg_b1, mp2_msg_W2, mp2_msg_b2, mp2_upd_W1, mp2_upd_b1, mp2_upd_W2, mp2_upd_b2, cls_W1, cls_b1, cls_W2, cls_b2):
    raise NotImplementedError("write your pallas kernel here")


# speedup vs baseline: 1.8866x; 1.8866x over previous
import functools

import jax
import jax.numpy as jnp
from jax import lax
from jax.experimental import pallas as pl
from jax.experimental.pallas import tpu as pltpu
from jax.experimental.pallas import tpu_sc as plsc

N = 10000
E = 160000
G = 100
HID = 256
HALF = 128
E_PAD = 163840
CHK = 128
ROWS = E_PAD // CHK
NT = 10016
ST = 10240
DUMP = 10008
F32 = jnp.float32


def _mesh():
    return plsc.VectorSubcoreMesh(core_axis_name="c", subcore_axis_name="s")


def _sc_dx(ii2d, jj2d, pos_pad):
    @functools.partial(
        pl.kernel,
        out_type=jax.ShapeDtypeStruct((E_PAD, HALF), F32),
        mesh=_mesh(),
        scratch_types=[
            pltpu.VMEM((40, CHK), jnp.int32),
            pltpu.VMEM((CHK,), jnp.int32),
            pltpu.VMEM((CHK, HALF), F32),
            pltpu.VMEM((CHK, HALF), F32),
            pltpu.SemaphoreType.DMA,
            pltpu.SemaphoreType.DMA,
        ],
    )
    def k(ii_hbm, jj_hbm, pos_hbm, dx_out,
          ii_v, jj_c, pi_v, pj_v, sem1, sem2):
        c = lax.axis_index("c")
        s = lax.axis_index("s")
        w = c * 16 + s

        pltpu.sync_copy(ii_hbm.at[pl.ds(w * 40, 40)], ii_v)

        def chunk(kk, _):
            row = w * 40 + kk
            pltpu.sync_copy(jj_hbm.at[row], jj_c)
            d1 = pltpu.async_copy(pos_hbm.at[ii_v.at[kk]], pi_v, sem1)
            d2 = pltpu.async_copy(pos_hbm.at[jj_c], pj_v, sem2)
            d1.wait()
            d2.wait()

            def rowfn(r, _):
                for q in range(8):
                    sl = pl.ds(q * 16, 16)
                    pi_v[r, sl] = pj_v[r, sl] - pi_v[r, sl]
                return 0

            lax.fori_loop(0, CHK, rowfn, 0)
            pltpu.sync_copy(pi_v, dx_out.at[pl.ds(row * CHK, CHK)])
            return 0

        lax.fori_loop(0, 40, chunk, 0)

    return k(ii2d, jj2d, pos_pad)


def _sc_deg(ii2d):
    @functools.partial(
        pl.kernel,
        out_type=jax.ShapeDtypeStruct((2, ST, HALF), F32),
        mesh=_mesh(),
        scratch_types=[
            pltpu.VMEM((40, CHK), jnp.int32),
            pltpu.VMEM((CHK, HALF), F32),
            pltpu.VMEM_SHARED((ST, HALF), F32),
        ],
    )
    def k(ii_hbm, deg_out, ii_v, ones_v, deg_sh):
        c = lax.axis_index("c")
        s = lax.axis_index("s")
        w = c * 16 + s

        def fill(r, _):
            for q in range(8):
                ones_v[r, pl.ds(q * 16, 16)] = jnp.zeros((16,), F32)
            return 0

        lax.fori_loop(0, CHK, fill, 0)
        for z in range(5):
            pltpu.sync_copy(ones_v, deg_sh.at[pl.ds(s * 640 + z * CHK, CHK)])

        def fill1(r, _):
            for q in range(8):
                ones_v[r, pl.ds(q * 16, 16)] = jnp.full((16,), 1.0, F32)
            return 0

        lax.fori_loop(0, CHK, fill1, 0)
        plsc.subcore_barrier()

        pltpu.sync_copy(ii_hbm.at[pl.ds(w * 40, 40)], ii_v)

        def chunk(kk, _):
            pltpu.sync_copy(ones_v, deg_sh.at[ii_v.at[kk]], add=True)
            return 0

        lax.fori_loop(0, 40, chunk, 0)
        plsc.subcore_barrier()
        pltpu.sync_copy(deg_sh.at[pl.ds(s * 640, 640)],
                        deg_out.at[c].at[pl.ds(s * 640, 640)])

    return k(ii2d)


def _sc_edge_pass(ii2d, jj2d, A, B, C):
    @functools.partial(
        pl.kernel,
        out_type=jax.ShapeDtypeStruct((2, ST, HALF), F32),
        mesh=_mesh(),
        scratch_types=[
            pltpu.VMEM((80, CHK), jnp.int32),
            pltpu.VMEM((CHK,), jnp.int32),
            pltpu.VMEM((CHK, HALF), F32),
            pltpu.VMEM((CHK, HALF), F32),
            pltpu.VMEM_SHARED((ST, HALF), F32),
            pltpu.SemaphoreType.DMA,
            pltpu.SemaphoreType.DMA,
            pltpu.SemaphoreType.DMA,
        ],
    )
    def k(ii_hbm, jj_hbm, a_hbm, b_hbm, c_hbm, s_out,
          ii_v, jj_c, a_v, b_v, s_sh, sem1, sem2, sem3):
        c = lax.axis_index("c")
        s = lax.axis_index("s")

        def zrow(r, _):
            for q in range(8):
                a_v[r, pl.ds(q * 16, 16)] = jnp.zeros((16,), F32)
            return 0

        lax.fori_loop(0, CHK, zrow, 0)
        for z in range(5):
            pltpu.sync_copy(a_v, s_sh.at[pl.ds(s * 640 + z * CHK, CHK)])
        plsc.subcore_barrier()

        pltpu.sync_copy(ii_hbm.at[pl.ds(s * 80, 80)], ii_v)

        def chunk(kk, _):
            row = s * 80 + kk
            pltpu.sync_copy(jj_hbm.at[row], jj_c)
            d1 = pltpu.async_copy(a_hbm.at[c].at[ii_v.at[kk]], a_v, sem1)
            d2 = pltpu.async_copy(b_hbm.at[c].at[jj_c], b_v, sem2)
            d1.wait()
            d2.wait()

            def addfn(r, _):
                for q in range(8):
                    sl = pl.ds(q * 16, 16)
                    a_v[r, sl] = a_v[r, sl] + b_v[r, sl]
                return 0

            lax.fori_loop(0, CHK, addfn, 0)
            d3 = pltpu.async_copy(c_hbm.at[c].at[pl.ds(row * CHK, CHK)],
                                  b_v, sem3)
            d3.wait()

            def relufn(r, _):
                for q in range(8):
                    sl = pl.ds(q * 16, 16)
                    a_v[r, sl] = jnp.maximum(a_v[r, sl] + b_v[r, sl], 0.0)
                return 0

            lax.fori_loop(0, CHK, relufn, 0)
            pltpu.sync_copy(a_v, s_sh.at[ii_v.at[kk]], add=True)
            return 0

        lax.fori_loop(0, 80, chunk, 0)
        plsc.subcore_barrier()
        pltpu.sync_copy(s_sh.at[pl.ds(s * 640, 640)],
                        s_out.at[c].at[pl.ds(s * 640, 640)])

    return k(ii2d, jj2d, A, B, C)


def _tc_fold_weights(ee_W2, Wc_all, ee_b2_row, b1cat_row):
    def body(w2_ref, wc_ref, b2_ref, b1_ref, m_ref, b_ref):
        m_ref[...] = jnp.dot(w2_ref[...], wc_ref[...],
                             preferred_element_type=F32)
        b_ref[...] = jnp.dot(b2_ref[...], wc_ref[...],
                             preferred_element_type=F32) + b1_ref[...]

    return pl.pallas_call(
        body,
        out_shape=(
            jax.ShapeDtypeStruct((HID, 3 * HID), F32),
            jax.ShapeDtypeStruct((1, 3 * HID), F32),
        ),
    )(ee_W2, Wc_all, ee_b2_row, b1cat_row)


def _tc_geom(dxa, ee_W1p, ee_b1_row, M_all, b_all):
    BLK = 512

    def body(dx_ref, w1_ref, b1_ref, m_ref, ba_ref, c0, c1, c2):
        dx = dx_ref[...]
        d0 = dx[:, 0:1]
        d1 = dx[:, 1:2]
        d2 = dx[:, 2:3]
        s3 = jnp.sqrt(d0 * d0 + d1 * d1 + d2 * d2)
        sxy = jnp.sqrt(d0 * d0 + d1 * d1)
        g = (s3 * w1_ref[0:1, :] + sxy * w1_ref[1:2, :] + d2 * w1_ref[2:3, :]
             + b1_ref[...])
        r = jnp.maximum(g, 0.0)
        call = jnp.dot(r, m_ref[...], preferred_element_type=F32) + ba_ref[...]
        for l, o in enumerate((c0, c1, c2)):
            o[0] = call[:, l * 256:l * 256 + 128]
            o[1] = call[:, l * 256 + 128:l * 256 + 256]

    outs = pl.pallas_call(
        body,
        grid=(E_PAD // BLK,),
        in_specs=[
            pl.BlockSpec((BLK, HALF), lambda e: (e, 0)),
            pl.BlockSpec((8, HID), lambda e: (0, 0)),
            pl.BlockSpec((1, HID), lambda e: (0, 0)),
            pl.BlockSpec((HID, 3 * HID), lambda e: (0, 0)),
            pl.BlockSpec((1, 3 * HID), lambda e: (0, 0)),
        ],
        out_specs=[
            pl.BlockSpec((2, BLK, HALF), lambda e: (0, e, 0)),
            pl.BlockSpec((2, BLK, HALF), lambda e: (0, e, 0)),
            pl.BlockSpec((2, BLK, HALF), lambda e: (0, e, 0)),
        ],
        out_shape=[jax.ShapeDtypeStruct((2, E_PAD, HALF), F32)] * 3,
    )(dxa, ee_W1p, ee_b1_row, M_all, b_all)
    return outs


def _tc_node_embed(nf_pad, ne_W1p, ne_b1_row, ne_W2, ne_b2_row):
    def body(nf_ref, w1_ref, b1_ref, w2_ref, b2_ref, h_ref):
        hh = jnp.maximum(
            jnp.dot(nf_ref[...], w1_ref[...], preferred_element_type=F32)
            + b1_ref[...], 0.0)
        h_ref[...] = (jnp.dot(hh, w2_ref[...], preferred_element_type=F32)
                      + b2_ref[...])

    return pl.pallas_call(
        body,
        out_shape=jax.ShapeDtypeStruct((N, HID), F32),
    )(nf_pad, ne_W1p, ne_b1_row, ne_W2, ne_b2_row)


def _tc_pre(h, W1a, W1b):
    def body(h_ref, wa_ref, wb_ref, a_ref, b_ref):
        pa = jnp.dot(h_ref[...], wa_ref[...], preferred_element_type=F32)
        pb = jnp.dot(h_ref[...], wb_ref[...], preferred_element_type=F32)
        a_ref[0, pl.ds(0, N), :] = pa
        b_ref[0, pl.ds(0, N), :] = pb
        a_ref[0, pl.ds(N, NT - N), :] = jnp.zeros((NT - N, HALF), F32)
        b_ref[0, pl.ds(N, NT - N), :] = jnp.zeros((NT - N, HALF), F32)

    return pl.pallas_call(
        body,
        grid=(2,),
        in_specs=[
            pl.BlockSpec((N, HID), lambda hf: (0, 0)),
            pl.BlockSpec((HID, HALF), lambda hf: (0, hf)),
            pl.BlockSpec((HID, HALF), lambda hf: (0, hf)),
        ],
        out_specs=[
            pl.BlockSpec((1, NT, HALF), lambda hf: (hf, 0, 0)),
            pl.BlockSpec((1, NT, HALF), lambda hf: (hf, 0, 0)),
        ],
        out_shape=[jax.ShapeDtypeStruct((2, NT, HALF), F32)] * 2,
    )(h, W1a, W1b)


def _tc_update(h, S, deg, W2, b2_row, U1a, U1b, ub1_row, U2, ub2_row):
    BLK = 2000

    def body(h_ref, s_ref, d_ref, w2_ref, b2_ref, ua_ref, ub_ref, ub1_ref,
             u2_ref, ub2_ref, o_ref):
        hcur = h_ref[...]
        degt = d_ref[0, :, 0:1] + d_ref[1, :, 0:1]
        agg = (jnp.dot(s_ref[0], w2_ref[0:HALF, :],
                       preferred_element_type=F32)
               + jnp.dot(s_ref[1], w2_ref[HALF:HID, :],
                         preferred_element_type=F32)
               + degt * b2_ref[...])
        uh = jnp.maximum(
            jnp.dot(hcur, ua_ref[...], preferred_element_type=F32)
            + jnp.dot(agg, ub_ref[...], preferred_element_type=F32)
            + ub1_ref[...], 0.0)
        o_ref[...] = hcur + (jnp.dot(uh, u2_ref[...],
                                     preferred_element_type=F32)
                             + ub2_ref[...])

    return pl.pallas_call(
        body,
        grid=(N // BLK,),
        in_specs=[
            pl.BlockSpec((BLK, HID), lambda n: (n, 0)),
            pl.BlockSpec((2, BLK, HALF), lambda n: (0, n, 0)),
            pl.BlockSpec((2, BLK, HALF), lambda n: (0, n, 0)),
            pl.BlockSpec((HID, HID), lambda n: (0, 0)),
            pl.BlockSpec((1, HID), lambda n: (0, 0)),
            pl.BlockSpec((HID, HID), lambda n: (0, 0)),
            pl.BlockSpec((HID, HID), lambda n: (0, 0)),
            pl.BlockSpec((1, HID), lambda n: (0, 0)),
            pl.BlockSpec((HID, HID), lambda n: (0, 0)),
            pl.BlockSpec((1, HID), lambda n: (0, 0)),
        ],
        out_specs=pl.BlockSpec((BLK, HID), lambda n: (n, 0)),
        out_shape=jax.ShapeDtypeStruct((N, HID), F32),
    )(h, S, deg, W2, b2_row, U1a, U1b, ub1_row, U2, ub2_row)


def _tc_pool_cls(h, batch_row, cls_W1, cls_b1_row, cls_W2p, cls_b2p):
    def body(h_ref, b_ref, w1_ref, b1_ref, w2_ref, b2_ref, o_ref):
        ids = lax.broadcasted_iota(jnp.int32, (HALF, N), 0)
        oht = (b_ref[...] == ids).astype(F32)
        ge = jnp.dot(oht, h_ref[...], preferred_element_type=F32)
        cnt = jnp.sum(oht, axis=1, keepdims=True)
        gem = ge / jnp.maximum(cnt, 1.0)
        hh = jnp.maximum(
            jnp.dot(gem, w1_ref[...], preferred_element_type=F32)
            + b1_ref[...], 0.0)
        o_ref[...] = (jnp.dot(hh, w2_ref[...], preferred_element_type=F32)
                      + b2_ref[...])

    return pl.pallas_call(
        body,
        out_shape=jax.ShapeDtypeStruct((HALF, HALF), F32),
    )(h, batch_row, cls_W1, cls_b1_row, cls_W2p, cls_b2p)


def kernel(positions, node_features, edge_index, batch,
           ne_W1, ne_b1, ne_W2, ne_b2, ee_W1, ee_b1, ee_W2, ee_b2,
           mp0_msg_W1, mp0_msg_b1, mp0_msg_W2, mp0_msg_b2,
           mp0_upd_W1, mp0_upd_b1, mp0_upd_W2, mp0_upd_b2,
           mp1_msg_W1, mp1_msg_b1, mp1_msg_W2, mp1_msg_b2,
           mp1_upd_W1, mp1_upd_b1, mp1_upd_W2, mp1_upd_b2,
           mp2_msg_W1, mp2_msg_b1, mp2_msg_W2, mp2_msg_b2,
           mp2_upd_W1, mp2_upd_b1, mp2_upd_W2, mp2_upd_b2,
           cls_W1, cls_b1, cls_W2, cls_b2):
    msg_W1 = (mp0_msg_W1, mp1_msg_W1, mp2_msg_W1)
    msg_b1 = (mp0_msg_b1, mp1_msg_b1, mp2_msg_b1)
    msg_W2 = (mp0_msg_W2, mp1_msg_W2, mp2_msg_W2)
    msg_b2 = (mp0_msg_b2, mp1_msg_b2, mp2_msg_b2)
    upd_W1 = (mp0_upd_W1, mp1_upd_W1, mp2_upd_W1)
    upd_b1 = (mp0_upd_b1, mp1_upd_b1, mp2_upd_b1)
    upd_W2 = (mp0_upd_W2, mp1_upd_W2, mp2_upd_W2)
    upd_b2 = (mp0_upd_b2, mp1_upd_b2, mp2_upd_b2)

    i_idx = edge_index[0]
    j_idx = edge_index[1]
    pad = E_PAD - E
    ii2d = jnp.concatenate(
        [i_idx, jnp.full((pad,), DUMP, jnp.int32)]).reshape(ROWS, CHK)
    jj2d = jnp.concatenate(
        [j_idx, jnp.zeros((pad,), jnp.int32)]).reshape(ROWS, CHK)
    pos_pad = jnp.pad(positions, ((0, NT - N), (0, HALF - 3)))

    dxa = _sc_dx(ii2d, jj2d, pos_pad)
    deg = _sc_deg(ii2d)

    Wc_all = jnp.concatenate([w[2 * HID:3 * HID] for w in msg_W1], axis=1)
    b1cat = jnp.concatenate(msg_b1).reshape(1, 3 * HID)
    M_all, b_all = _tc_fold_weights(ee_W2, Wc_all, ee_b2.reshape(1, HID),
                                    b1cat)

    ee_W1p = jnp.pad(ee_W1, ((0, 5), (0, 0)))
    C0, C1, C2 = _tc_geom(dxa, ee_W1p, ee_b1.reshape(1, HID), M_all, b_all)
    C = (C0, C1, C2)

    nf_pad = jnp.pad(node_features, ((0, 0), (0, 6)))
    ne_W1p = jnp.pad(ne_W1, ((0, 6), (0, 0)))
    h = _tc_node_embed(nf_pad, ne_W1p, ne_b1.reshape(1, HID), ne_W2,
                       ne_b2.reshape(1, HID))

    for l in range(3):
        A, B = _tc_pre(h, msg_W1[l][0:HID], msg_W1[l][HID:2 * HID])
        S = _sc_edge_pass(ii2d, jj2d, A, B, C[l])
        h = _tc_update(h, S, deg, msg_W2[l], msg_b2[l].reshape(1, HID),
                       upd_W1[l][0:HID], upd_W1[l][HID:2 * HID],
                       upd_b1[l].reshape(1, HID), upd_W2[l],
                       upd_b2[l].reshape(1, HID))

    cls_W2p = jnp.pad(cls_W2, ((0, 0), (0, HALF - 1)))
    cls_b2p = jnp.pad(cls_b2.reshape(1, 1), ((0, 0), (0, HALF - 1)))
    out = _tc_pool_cls(h, batch.reshape(1, N), cls_W1,
                       cls_b1.reshape(1, HID), cls_W2p, cls_b2p)
    return out[:G, 0]
